# baseline (speedup 1.0000x reference)
import functools

import jax
import jax.numpy as jnp
from jax import lax
from jax.experimental import pallas as pl
from jax.experimental.pallas import tpu as pltpu
from jax.experimental.pallas import tpu_sc as plsc

D = 128
NC = 2
NS = 16
NW = NC * NS
BLK = 128


def _sc_segment_sum(table_rows, acc_rows, nb):
    assert nb % 4 == 0
    seg = nb // 2
    rpt = (acc_rows // (NS * 8)) * 8
    tail = acc_rows - NS * rpt
    mesh = plsc.VectorSubcoreMesh(
        core_axis_name="c", subcore_axis_name="s", num_cores=NC, num_subcores=NS
    )

    @functools.partial(
        pl.kernel,
        out_type=jax.ShapeDtypeStruct((NC, acc_rows, D), jnp.float32),
        mesh=mesh,
        scratch_types=[
            pltpu.VMEM((seg, 2, BLK), jnp.int32),
            pltpu.VMEM((2 * BLK, D), jnp.float32),
            pltpu.VMEM_SHARED((acc_rows, D), jnp.float32),
            pltpu.SemaphoreType.DMA,
            pltpu.SemaphoreType.DMA,
            pltpu.SemaphoreType.DMA,
            pltpu.SemaphoreType.DMA,
        ],
    )
    def sc_kernel(tab_hbm, idx_hbm, zeros_hbm, out_hbm,
                  idx_v, rows_v, acc_s, g0, g1, s0, s1):
        c = lax.axis_index("c")
        s = lax.axis_index("s")
        wid = c * NS + s
        r0 = s * rpt
        gsems = (g0, g1)
        ssems = (s0, s1)
        pltpu.sync_copy(zeros_hbm.at[pl.ds(r0, rpt)], acc_s.at[pl.ds(r0, rpt)])
        if tail:
            pl.when(s == 0)(lambda: pltpu.sync_copy(
                zeros_hbm.at[pl.ds(NS * rpt, tail)],
                acc_s.at[pl.ds(NS * rpt, tail)]))
        plsc.subcore_barrier()

        def wait_gather(b, q):
            pltpu.make_async_copy(tab_hbm.at[idx_v.at[b, 0]],
                                  rows_v.at[pl.ds(q * BLK, BLK)],
                                  gsems[q]).wait()

        def wait_scatter(b, q):
            pltpu.make_async_copy(rows_v.at[pl.ds(q * BLK, BLK)],
                                  acc_s.at[idx_v.at[b, 1]], ssems[q]).wait()

        def fire_gather(b, q):
            pltpu.async_copy(tab_hbm.at[idx_v.at[b, 0]],
                             rows_v.at[pl.ds(q * BLK, BLK)], gsems[q])

        def fire_scatter(b, q):
            pltpu.async_copy(rows_v.at[pl.ds(q * BLK, BLK)],
                             acc_s.at[idx_v.at[b, 1]], ssems[q], add=True)

        def run_segment(sg):
            pltpu.sync_copy(idx_hbm.at[wid, pl.ds(sg * seg, seg)], idx_v)

            def body(b, carry):
                pltpu.async_copy(tab_hbm.at[idx_v.at[b, 0]],
                                 rows_v.at[pl.ds(0, BLK)], g0).wait()
                pltpu.async_copy(rows_v.at[pl.ds(0, BLK)],
                                 acc_s.at[idx_v.at[b, 1]], s0, add=True).wait()
                return carry

            lax.fori_loop(0, seg, body, 0)

        run_segment(0)
        run_segment(1)
        plsc.subcore_barrier()
        pltpu.sync_copy(acc_s.at[pl.ds(r0, rpt)], out_hbm.at[c, pl.ds(r0, rpt)])
        if tail:
            pl.when(s == 0)(lambda: pltpu.sync_copy(
                acc_s.at[pl.ds(NS * rpt, tail)],
                out_hbm.at[c, pl.ds(NS * rpt, tail)]))

    return sc_kernel


def _dense_mid(p_ref, wlT_ref, bl_ref, whT_ref, wt_ref, cf_ref, y_ref):
    hc = p_ref[0] + p_ref[1]
    cembs = jnp.maximum(
        jnp.dot(hc, wlT_ref[...], preferred_element_type=jnp.float32)
        + bl_ref[...], 0.0)
    y_ref[...] = (
        jnp.dot(cembs, whT_ref[...], preferred_element_type=jnp.float32)
        + cf_ref[...] * wt_ref[...])


def _dense_out(p_ref, bo_ref, o_ref, n_out):
    o_ref[...] = jnp.maximum(p_ref[0, :n_out] + p_ref[1, :n_out] + bo_ref[...], 0.0)


def kernel(literal_feat, clause_feat, W_l2c, b_l2c, W_c2l, b_c2l, lit_idx, clause_idx):
    n_lit, _ = literal_feat.shape
    n_clause = clause_feat.shape[0]
    e = lit_idx.shape[0]

    c_pad = ((n_clause + 1 + NS - 1) // NS) * NS
    l_pad = ((n_lit + 1 + NS - 1) // NS) * NS

    nb = -(-e // (NW * BLK))
    nb = max(4, -(-nb // 4) * 4)
    e_pad = NW * nb * BLK
    li = jnp.concatenate(
        [lit_idx.astype(jnp.int32), jnp.full((e_pad - e,), n_lit, jnp.int32)]
    ).reshape(NW, nb, 1, BLK)
    ci = jnp.concatenate(
        [clause_idx.astype(jnp.int32), jnp.full((e_pad - e,), n_clause, jnp.int32)]
    ).reshape(NW, nb, 1, BLK)
    idx_p1 = jnp.concatenate([li, ci], axis=2)
    idx_p2 = jnp.concatenate([ci, li], axis=2)

    lit_tab = jnp.concatenate(
        [literal_feat, jnp.zeros((16, D), jnp.float32)], axis=0)
    zeros = jnp.zeros((l_pad, D), jnp.float32)

    part_c = _sc_segment_sum(lit_tab.shape[0], c_pad, nb)(lit_tab, idx_p1, zeros)

    wlT = W_l2c.T
    whT = W_c2l[:, :D].T
    wt = W_c2l[:, D].reshape(1, D)
    cf = jnp.concatenate(
        [clause_feat.astype(jnp.float32),
         jnp.zeros((c_pad - n_clause, 1), jnp.float32)], axis=0)
    y = pl.pallas_call(
        _dense_mid,
        out_shape=jax.ShapeDtypeStruct((c_pad, D), jnp.float32),
    )(part_c, wlT, b_l2c.reshape(1, D), whT, wt, cf)

    part_l = _sc_segment_sum(c_pad, l_pad, nb)(y, idx_p2, zeros)

    lembs = pl.pallas_call(
        functools.partial(_dense_out, n_out=n_lit),
        out_shape=jax.ShapeDtypeStruct((n_lit, D), jnp.float32),
    )(part_l, b_c2l.reshape(1, D))
    return lembs


# speedup vs baseline: 1.5169x; 1.5169x over previous
import functools

import jax
import jax.numpy as jnp
from jax import lax
from jax.experimental import pallas as pl
from jax.experimental.pallas import tpu as pltpu
from jax.experimental.pallas import tpu_sc as plsc

D = 128
NC = 2
NS = 16
NW = NC * NS
BLK = 128


def _sc_segment_sum(table_rows, acc_rows, nb):
    rpt = acc_rows // NS
    mesh = plsc.VectorSubcoreMesh(
        core_axis_name="c", subcore_axis_name="s", num_cores=NC, num_subcores=NS
    )

    @functools.partial(
        pl.kernel,
        out_type=jax.ShapeDtypeStruct((NC, acc_rows, D), jnp.float32),
        mesh=mesh,
        scratch_types=[
            pltpu.VMEM((nb, BLK), jnp.int32),
            pltpu.VMEM((nb, BLK), jnp.int32),
            pltpu.VMEM((BLK, D), jnp.float32),
            pltpu.VMEM_SHARED((acc_rows, D), jnp.float32),
            pltpu.SemaphoreType.DMA,
            pltpu.SemaphoreType.DMA,
        ],
    )
    def sc_kernel(tab_hbm, gidx_hbm, sidx_hbm, zeros_hbm, out_hbm,
                  gidx_v, sidx_v, rows_v, acc_s, gsem, ssem):
        c = lax.axis_index("c")
        s = lax.axis_index("s")
        wid = c * NS + s
        r0 = s * rpt
        pltpu.sync_copy(zeros_hbm.at[pl.ds(r0, rpt)], acc_s.at[pl.ds(r0, rpt)])
        pltpu.sync_copy(gidx_hbm.at[wid], gidx_v)
        pltpu.sync_copy(sidx_hbm.at[wid], sidx_v)
        plsc.subcore_barrier()

        def body(b, carry):
            pltpu.async_copy(tab_hbm.at[gidx_v.at[b]], rows_v, gsem).wait()
            pltpu.async_copy(rows_v, acc_s.at[sidx_v.at[b]], ssem, add=True).wait()
            return carry

        lax.fori_loop(0, nb, body, 0)
        plsc.subcore_barrier()
        pltpu.sync_copy(acc_s.at[pl.ds(r0, rpt)], out_hbm.at[c, pl.ds(r0, rpt)])

    return sc_kernel


def _dense_mid(p_ref, wlT_ref, bl_ref, whT_ref, wt_ref, cf_ref, y_ref):
    hc = p_ref[0] + p_ref[1]
    cembs = jnp.maximum(
        jnp.dot(hc, wlT_ref[...], preferred_element_type=jnp.float32)
        + bl_ref[...], 0.0)
    y_ref[...] = (
        jnp.dot(cembs, whT_ref[...], preferred_element_type=jnp.float32)
        + cf_ref[...] * wt_ref[...])


def _dense_out(p_ref, bo_ref, o_ref, n_out):
    o_ref[...] = jnp.maximum(p_ref[0, :n_out] + p_ref[1, :n_out] + bo_ref[...], 0.0)


def kernel(literal_feat, clause_feat, W_l2c, b_l2c, W_c2l, b_c2l, lit_idx, clause_idx):
    n_lit, _ = literal_feat.shape
    n_clause = clause_feat.shape[0]
    e = lit_idx.shape[0]

    c_pad = ((n_clause + 1 + NS * 8 - 1) // (NS * 8)) * (NS * 8)
    l_pad = ((n_lit + 1 + NS * 8 - 1) // (NS * 8)) * (NS * 8)

    nb = -(-e // (NW * BLK))
    e_pad = NW * nb * BLK
    li = jnp.concatenate(
        [lit_idx.astype(jnp.int32), jnp.full((e_pad - e,), n_lit, jnp.int32)]
    ).reshape(NW, nb, BLK)
    ci = jnp.concatenate(
        [clause_idx.astype(jnp.int32), jnp.full((e_pad - e,), n_clause, jnp.int32)]
    ).reshape(NW, nb, BLK)

    lit_tab = jnp.concatenate(
        [literal_feat, jnp.zeros((16, D), jnp.float32)], axis=0)
    zeros = jnp.zeros((l_pad, D), jnp.float32)

    part_c = _sc_segment_sum(lit_tab.shape[0], c_pad, nb)(lit_tab, li, ci, zeros)

    wlT = W_l2c.T
    whT = W_c2l[:, :D].T
    wt = W_c2l[:, D].reshape(1, D)
    cf = jnp.concatenate(
        [clause_feat.astype(jnp.float32),
         jnp.zeros((c_pad - n_clause, 1), jnp.float32)], axis=0)
    y = pl.pallas_call(
        _dense_mid,
        out_shape=jax.ShapeDtypeStruct((c_pad, D), jnp.float32),
    )(part_c, wlT, b_l2c.reshape(1, D), whT, wt, cf)

    part_l = _sc_segment_sum(c_pad, l_pad, nb)(y, ci, li, zeros)

    lembs = pl.pallas_call(
        functools.partial(_dense_out, n_out=n_lit),
        out_shape=jax.ShapeDtypeStruct((n_lit, D), jnp.float32),
    )(part_l, b_c2l.reshape(1, D))
    return lembs
